# baseline (speedup 1.0000x reference)
import functools

import jax
import jax.numpy as jnp
from jax import lax
from jax.experimental import pallas as pl
from jax.experimental.pallas import tpu as pltpu
from jax.experimental.pallas import tpu_sc as plsc

N = 100000
NENC = 102400
TPAD = 100096
E = 3200000
NC = 2
NS = 16
NW = NC * NS
EW = 64
KB = 8
NSLOT = 3
PW0 = 1320
PW1 = 1848
EPAD = NS * (PW0 + PW1) * EW
TR_E = EPAD // EW
PAD_T = TPAD - N
ROWS_PER_SUB = TPAD // NS
R2D = TPAD * 16 // 128


def _mesh():
    return plsc.VectorSubcoreMesh(core_axis_name="c", subcore_axis_name="s",
                                  num_cores=NC, num_subcores=NS)


_SC_PARAMS = pltpu.CompilerParams(use_tc_tiling_on_sc=False)


def _encode_sc(ids2d, emb_rep):
    @functools.partial(
        pl.kernel,
        out_type=jax.ShapeDtypeStruct((NENC, 16), jnp.float32),
        mesh=_mesh(),
        compiler_params=_SC_PARAMS,
        scratch_types=[
            pltpu.VMEM((32, 128), jnp.int32),
            pltpu.VMEM((32, 128, 16), jnp.float32),
            pltpu.SemaphoreType.DMA,
        ],
    )
    def enc(ids_hbm, emb_hbm, h_hbm, eidx, erows, sem):
        w = lax.axis_index("s") * NC + lax.axis_index("c")

        @pl.when(w < 25)
        def _():
            pltpu.sync_copy(ids_hbm.at[pl.ds(w * 32, 32)], eidx)
            for half in range(2):
                descs = [pltpu.async_copy(emb_hbm.at[eidx.at[half * 16 + j]],
                                          erows.at[half * 16 + j], sem)
                         for j in range(16)]
                for d in descs:
                    d.wait()
            for j in range(32):
                pltpu.sync_copy(erows.at[j],
                                h_hbm.at[pl.ds(w * 4096 + j * 128, 128)])

    return enc(ids2d, emb_rep)


def _edge_agg_sc(h, se2d, de2d, zeros_hbm):
    @functools.partial(
        pl.kernel,
        out_type=jax.ShapeDtypeStruct((NC, TPAD, 16), jnp.float32),
        mesh=_mesh(),
        compiler_params=_SC_PARAMS,
        scratch_types=[
            pltpu.VMEM_SHARED((TPAD, 16), jnp.float32),
            pltpu.VMEM((NSLOT, KB, EW), jnp.int32),
            pltpu.VMEM((NSLOT, KB, EW), jnp.int32),
            pltpu.VMEM((NSLOT, KB, EW, 16), jnp.float32),
            pltpu.SemaphoreType.DMA,
            pltpu.SemaphoreType.DMA,
        ],
    )
    def edge(h_hbm, se_hbm, de_hbm, z_hbm, out_hbm,
             agg_sh, sidx, didx, rows, gsem, ssem):
        c = lax.axis_index("c")
        s = lax.axis_index("s")
        w_rows = jnp.where(c == 0, s * PW0, NS * PW0 + s * PW1)
        nk = jnp.where(c == 0, PW0 // (NSLOT * KB), PW1 // (NSLOT * KB))
        pltpu.sync_copy(z_hbm, agg_sh.at[pl.ds(s * ROWS_PER_SUB, ROWS_PER_SUB)])
        plsc.subcore_barrier()

        def body(k, carry):
            base0 = w_rows + k * (NSLOT * KB)

            def load_fire(slot):
                b = base0 + slot * KB
                pltpu.sync_copy(se_hbm.at[pl.ds(b, KB)], sidx.at[slot])
                pltpu.sync_copy(de_hbm.at[pl.ds(b, KB)], didx.at[slot])
                return [pltpu.async_copy(h_hbm.at[sidx.at[slot, j]],
                                         rows.at[slot, j], gsem)
                        for j in range(KB)]

            def scat(slot):
                return [pltpu.async_copy(rows.at[slot, j],
                                         agg_sh.at[didx.at[slot, j]],
                                         ssem, add=True)
                        for j in range(KB)]

            gath = [None] * NSLOT
            scas = []
            gath[0] = load_fire(0)
            gath[1] = load_fire(1)
            for slot in range(NSLOT):
                for d in gath[slot]:
                    d.wait()
                scas.extend(scat(slot))
                nxt = slot + 2
                if nxt < NSLOT:
                    gath[nxt] = load_fire(nxt)
            for d in scas:
                d.wait()
            return carry

        lax.fori_loop(0, nk, body, 0)
        plsc.subcore_barrier()
        pltpu.sync_copy(agg_sh.at[pl.ds(s * ROWS_PER_SUB, ROWS_PER_SUB)],
                        out_hbm.at[c, pl.ds(s * ROWS_PER_SUB, ROWS_PER_SUB)])

    return edge(h, se2d, de2d, zeros_hbm)


def _leaky(x):
    return jnp.where(x >= 0, x, 0.01 * x)


def _block_diag_wu(w, u):
    wu = jnp.dot(w, u, preferred_element_type=jnp.float32)
    t = jnp.tile(wu, (8, 8))
    r = lax.broadcasted_iota(jnp.int32, (128, 128), 0) // 16
    c = lax.broadcasted_iota(jnp.int32, (128, 128), 1) // 16
    return jnp.where(r == c, t, 0.0)


def _update_tc(h2d, agg2d, w, u):
    def upd(h_ref, a_ref, w_ref, u_ref, o_ref):
        bd = _block_diag_wu(w_ref[...], u_ref[...])
        x = a_ref[0] + a_ref[1]
        o_ref[...] = _leaky(
            h_ref[...] + jnp.dot(x, bd, preferred_element_type=jnp.float32))

    return pl.pallas_call(
        upd, out_shape=jax.ShapeDtypeStruct((R2D, 128), jnp.float32),
    )(h2d, agg2d, w, u)


def _final_tc(h2d, agg2d, w, u, bexp, fp, l1a, l1b, b1, l2w, b2):
    def fin(h_ref, a_ref, w_ref, u_ref, b_ref, fp_ref,
            l1a_ref, l1b_ref, b1_ref, l2w_ref, b2_ref, o_ref):
        bd = _block_diag_wu(w_ref[...], u_ref[...])
        x = a_ref[0] + a_ref[1]
        h3 = _leaky(
            h_ref[...] + jnp.dot(x, bd, preferred_element_type=jnp.float32))
        b = b_ref[...]
        cols = []
        for g in range(16):
            m = (b == g).astype(jnp.float32)
            cols.append(jnp.sum(h3 * m, axis=0, keepdims=True))
        p128 = jnp.concatenate(cols, axis=0)
        fi = lax.broadcasted_iota(jnp.int32, (128, 16), 0) % 16
        fj = lax.broadcasted_iota(jnp.int32, (128, 16), 1)
        fold = (fi == fj).astype(jnp.float32)
        pooled = jnp.dot(p128, fold, preferred_element_type=jnp.float32)
        a1 = _leaky(jnp.dot(pooled, l1a_ref[...],
                            preferred_element_type=jnp.float32)
                    + jnp.dot(fp_ref[...], l1b_ref[...],
                              preferred_element_type=jnp.float32)
                    + b1_ref[...])
        o_ref[...] = (jnp.dot(a1, l2w_ref[...],
                              preferred_element_type=jnp.float32)
                      + b2_ref[...])

    return pl.pallas_call(
        fin, out_shape=jax.ShapeDtypeStruct((16, 1), jnp.float32),
    )(h2d, agg2d, w, u, bexp, fp, l1a, l1b, b1, l2w, b2)


def kernel(x_atm_species, edge_index, x_atm_batch, forcepair,
           emb, conv_W, conv_U, l1_W, l1_b, l2_W, l2_b):
    species = x_atm_species.astype(jnp.int32)
    src = edge_index[0].astype(jnp.int32)
    dst = edge_index[1].astype(jnp.int32)
    batch = x_atm_batch.astype(jnp.int32)

    pad_e = EPAD - E
    ids_enc = (jnp.concatenate([species, jnp.zeros((NENC - N,), jnp.int32)])
               + 10 * (jnp.arange(NENC, dtype=jnp.int32) % 64))
    ids2d = ids_enc.reshape(NENC // 128, 128)
    emb_rep = jnp.tile(emb.astype(jnp.float32), (64, 1))

    se2d = jnp.concatenate(
        [src, jnp.zeros((pad_e,), jnp.int32)]).reshape(TR_E, EW)
    dpad = N + (jnp.arange(pad_e, dtype=jnp.int32) % PAD_T)
    de2d = jnp.concatenate([dst, dpad]).reshape(TR_E, EW)
    zeros_hbm = jnp.zeros((ROWS_PER_SUB, 16), jnp.float32)

    bexp = jnp.repeat(
        jnp.concatenate([batch, jnp.full((TPAD - N,), 16, jnp.int32)]),
        16).reshape(R2D, 128)

    h = _encode_sc(ids2d, emb_rep)[:TPAD]
    for i in range(2):
        agg = _edge_agg_sc(h, se2d, de2d, zeros_hbm)
        h = _update_tc(h.reshape(R2D, 128),
                       agg.reshape(NC, R2D, 128),
                       conv_W[i], conv_U[i]).reshape(TPAD, 16)
    agg = _edge_agg_sc(h, se2d, de2d, zeros_hbm)
    return _final_tc(h.reshape(R2D, 128), agg.reshape(NC, R2D, 128),
                     conv_W[2], conv_U[2], bexp, forcepair,
                     l1_W[:16], l1_W[16:], l1_b.reshape(1, 16),
                     l2_W, l2_b.reshape(1, 1))


# speedup vs baseline: 1.1790x; 1.1790x over previous
import functools

import jax
import jax.numpy as jnp
from jax import lax
from jax.experimental import pallas as pl
from jax.experimental.pallas import tpu as pltpu
from jax.experimental.pallas import tpu_sc as plsc

N = 100000
NENC = 102400
TPAD = 100096
E = 3200000
NC = 2
NS = 16
NW = NC * NS
EW = 64
KB = 8
NSLOT = 3
PW0 = 1848
PW1 = 1320
EPAD = NS * (PW0 + PW1) * EW
TR_E = EPAD // EW
PAD_T = TPAD - N
ROWS_PER_SUB = TPAD // NS
R2D = TPAD * 16 // 128


def _mesh():
    return plsc.VectorSubcoreMesh(core_axis_name="c", subcore_axis_name="s",
                                  num_cores=NC, num_subcores=NS)


_SC_PARAMS = pltpu.CompilerParams(use_tc_tiling_on_sc=False)


def _encode_sc(ids2d, emb_rep):
    @functools.partial(
        pl.kernel,
        out_type=jax.ShapeDtypeStruct((NENC, 16), jnp.float32),
        mesh=_mesh(),
        compiler_params=_SC_PARAMS,
        scratch_types=[
            pltpu.VMEM((32, 128), jnp.int32),
            pltpu.VMEM((32, 128, 16), jnp.float32),
            pltpu.SemaphoreType.DMA,
        ],
    )
    def enc(ids_hbm, emb_hbm, h_hbm, eidx, erows, sem):
        w = lax.axis_index("s") * NC + lax.axis_index("c")

        @pl.when(w < 25)
        def _():
            pltpu.sync_copy(ids_hbm.at[pl.ds(w * 32, 32)], eidx)
            for half in range(2):
                descs = [pltpu.async_copy(emb_hbm.at[eidx.at[half * 16 + j]],
                                          erows.at[half * 16 + j], sem)
                         for j in range(16)]
                for d in descs:
                    d.wait()
            for j in range(32):
                pltpu.sync_copy(erows.at[j],
                                h_hbm.at[pl.ds(w * 4096 + j * 128, 128)])

    return enc(ids2d, emb_rep)


def _edge_agg_sc(h, se2d, de2d, zeros_hbm):
    @functools.partial(
        pl.kernel,
        out_type=jax.ShapeDtypeStruct((NC, TPAD, 16), jnp.float32),
        mesh=_mesh(),
        compiler_params=_SC_PARAMS,
        scratch_types=[
            pltpu.VMEM_SHARED((TPAD, 16), jnp.float32),
            pltpu.VMEM((NSLOT, KB, EW), jnp.int32),
            pltpu.VMEM((NSLOT, KB, EW), jnp.int32),
            pltpu.VMEM((NSLOT, KB, EW, 16), jnp.float32),
            pltpu.SemaphoreType.DMA,
            pltpu.SemaphoreType.DMA,
        ],
    )
    def edge(h_hbm, se_hbm, de_hbm, z_hbm, out_hbm,
             agg_sh, sidx, didx, rows, gsem, ssem):
        c = lax.axis_index("c")
        s = lax.axis_index("s")
        w_rows = jnp.where(c == 0, s * PW0, NS * PW0 + s * PW1)
        nk = jnp.where(c == 0, PW0 // (NSLOT * KB), PW1 // (NSLOT * KB))
        pltpu.sync_copy(z_hbm, agg_sh.at[pl.ds(s * ROWS_PER_SUB, ROWS_PER_SUB)])
        plsc.subcore_barrier()

        def body(k, carry):
            base0 = w_rows + k * (NSLOT * KB)

            def load_fire(slot):
                b = base0 + slot * KB
                pltpu.sync_copy(se_hbm.at[pl.ds(b, KB)], sidx.at[slot])
                pltpu.sync_copy(de_hbm.at[pl.ds(b, KB)], didx.at[slot])
                return [pltpu.async_copy(h_hbm.at[sidx.at[slot, j]],
                                         rows.at[slot, j], gsem)
                        for j in range(KB)]

            def scat(slot):
                return [pltpu.async_copy(rows.at[slot, j],
                                         agg_sh.at[didx.at[slot, j]],
                                         ssem, add=True)
                        for j in range(KB)]

            gath = [None] * NSLOT
            scas = []
            gath[0] = load_fire(0)
            gath[1] = load_fire(1)
            for slot in range(NSLOT):
                for d in gath[slot]:
                    d.wait()
                scas.extend(scat(slot))
                nxt = slot + 2
                if nxt < NSLOT:
                    gath[nxt] = load_fire(nxt)
            for d in scas:
                d.wait()
            return carry

        lax.fori_loop(0, nk, body, 0)
        plsc.subcore_barrier()
        pltpu.sync_copy(agg_sh.at[pl.ds(s * ROWS_PER_SUB, ROWS_PER_SUB)],
                        out_hbm.at[c, pl.ds(s * ROWS_PER_SUB, ROWS_PER_SUB)])

    return edge(h, se2d, de2d, zeros_hbm)


def _leaky(x):
    return jnp.where(x >= 0, x, 0.01 * x)


def _block_diag_wu(w, u):
    wu = jnp.dot(w, u, preferred_element_type=jnp.float32)
    t = jnp.tile(wu, (8, 8))
    r = lax.broadcasted_iota(jnp.int32, (128, 128), 0) // 16
    c = lax.broadcasted_iota(jnp.int32, (128, 128), 1) // 16
    return jnp.where(r == c, t, 0.0)


def _update_tc(h2d, agg2d, w, u):
    def upd(h_ref, a_ref, w_ref, u_ref, o_ref):
        bd = _block_diag_wu(w_ref[...], u_ref[...])
        x = a_ref[0] + a_ref[1]
        o_ref[...] = _leaky(
            h_ref[...] + jnp.dot(x, bd, preferred_element_type=jnp.float32))

    return pl.pallas_call(
        upd, out_shape=jax.ShapeDtypeStruct((R2D, 128), jnp.float32),
    )(h2d, agg2d, w, u)


def _final_tc(h2d, agg2d, w, u, bexp, fp, l1a, l1b, b1, l2w, b2):
    def fin(h_ref, a_ref, w_ref, u_ref, b_ref, fp_ref,
            l1a_ref, l1b_ref, b1_ref, l2w_ref, b2_ref, o_ref):
        bd = _block_diag_wu(w_ref[...], u_ref[...])
        x = a_ref[0] + a_ref[1]
        h3 = _leaky(
            h_ref[...] + jnp.dot(x, bd, preferred_element_type=jnp.float32))
        b = b_ref[...]
        cols = []
        for g in range(16):
            m = (b == g).astype(jnp.float32)
            cols.append(jnp.sum(h3 * m, axis=0, keepdims=True))
        p128 = jnp.concatenate(cols, axis=0)
        fi = lax.broadcasted_iota(jnp.int32, (128, 16), 0) % 16
        fj = lax.broadcasted_iota(jnp.int32, (128, 16), 1)
        fold = (fi == fj).astype(jnp.float32)
        pooled = jnp.dot(p128, fold, preferred_element_type=jnp.float32)
        a1 = _leaky(jnp.dot(pooled, l1a_ref[...],
                            preferred_element_type=jnp.float32)
                    + jnp.dot(fp_ref[...], l1b_ref[...],
                              preferred_element_type=jnp.float32)
                    + b1_ref[...])
        o_ref[...] = (jnp.dot(a1, l2w_ref[...],
                              preferred_element_type=jnp.float32)
                      + b2_ref[...])

    return pl.pallas_call(
        fin, out_shape=jax.ShapeDtypeStruct((16, 1), jnp.float32),
    )(h2d, agg2d, w, u, bexp, fp, l1a, l1b, b1, l2w, b2)


def kernel(x_atm_species, edge_index, x_atm_batch, forcepair,
           emb, conv_W, conv_U, l1_W, l1_b, l2_W, l2_b):
    species = x_atm_species.astype(jnp.int32)
    src = edge_index[0].astype(jnp.int32)
    dst = edge_index[1].astype(jnp.int32)
    batch = x_atm_batch.astype(jnp.int32)

    pad_e = EPAD - E
    ids_enc = (jnp.concatenate([species, jnp.zeros((NENC - N,), jnp.int32)])
               + 10 * (jnp.arange(NENC, dtype=jnp.int32) % 64))
    ids2d = ids_enc.reshape(NENC // 128, 128)
    emb_rep = jnp.tile(emb.astype(jnp.float32), (64, 1))

    se2d = jnp.concatenate(
        [src, jnp.zeros((pad_e,), jnp.int32)]).reshape(TR_E, EW)
    dpad = N + (jnp.arange(pad_e, dtype=jnp.int32) % PAD_T)
    de2d = jnp.concatenate([dst, dpad]).reshape(TR_E, EW)
    zeros_hbm = jnp.zeros((ROWS_PER_SUB, 16), jnp.float32)

    bexp = jnp.repeat(
        jnp.concatenate([batch, jnp.full((TPAD - N,), 16, jnp.int32)]),
        16).reshape(R2D, 128)

    h = _encode_sc(ids2d, emb_rep)[:TPAD]
    for i in range(2):
        agg = _edge_agg_sc(h, se2d, de2d, zeros_hbm)
        h = _update_tc(h.reshape(R2D, 128),
                       agg.reshape(NC, R2D, 128),
                       conv_W[i], conv_U[i]).reshape(TPAD, 16)
    agg = _edge_agg_sc(h, se2d, de2d, zeros_hbm)
    return _final_tc(h.reshape(R2D, 128), agg.reshape(NC, R2D, 128),
                     conv_W[2], conv_U[2], bexp, forcepair,
                     l1_W[:16], l1_W[16:], l1_b.reshape(1, 16),
                     l2_W, l2_b.reshape(1, 1))
